# Initial kernel scaffold; baseline (speedup 1.0000x reference)
import jax
import jax.numpy as jnp
from jax.experimental import pallas as pl


def kernel(x, w1, w2, w3, g1, b1, g2, b2, g3, b3):
    raise NotImplementedError("write your pallas kernel here")


# speedup vs baseline: 1.8979x; 1.8979x over previous
import jax
import jax.numpy as jnp
from jax import lax
from jax.experimental import pallas as pl
from jax.experimental.pallas import tpu as pltpu

_EPS = 1e-5
_VMEM_LIMIT = 32 * 1024 * 1024


def _stats_cols(y):
    return jnp.concatenate(
        [jnp.sum(y, axis=1, keepdims=True),
         jnp.sum(y * y, axis=1, keepdims=True)], axis=1)


def _finalize(st_ref, g_ref, b_ref, count):
    tot = jnp.sum(st_ref[...], axis=0)
    mean = tot[:, 0:1] / count
    var = jnp.maximum(tot[:, 1:2] / count - mean * mean, 0.0)
    scale = g_ref[...] * lax.rsqrt(var + _EPS)
    shift = b_ref[...] - mean * scale
    return scale, shift


def _conv1_kernel(x_ref, w_ref, y_ref, st_ref):
    xb = x_ref[...].astype(jnp.bfloat16)
    y = jnp.dot(w_ref[...], xb, preferred_element_type=jnp.float32)
    y_ref[...] = y.astype(jnp.bfloat16)
    st_ref[...] = _stats_cols(y)


def _make_conv2_kernel(W, HW, count):
    def body(y1_ref, st_ref, g_ref, b_ref, w_ref, y2_ref, st2_ref):
        C1 = y1_ref.shape[0]
        scale, shift = _finalize(st_ref, g_ref, b_ref, count)
        a = jnp.maximum(y1_ref[...].astype(jnp.float32) * scale + shift, 0.0)
        ab = a.astype(jnp.bfloat16)
        j = lax.broadcasted_iota(jnp.int32, (1, HW), 1) % W
        m_left = j > 0
        m_right = j < W - 1
        zero = jnp.bfloat16(0)
        slabs = []
        for di in range(3):
            for dj in range(3):
                s = (di - 1) * W + (dj - 1)
                if s == 0:
                    sh = ab
                elif s > 0:
                    sh = jnp.concatenate(
                        [ab[:, s:], jnp.zeros((C1, s), ab.dtype)], axis=1)
                else:
                    sh = jnp.concatenate(
                        [jnp.zeros((C1, -s), ab.dtype), ab[:, :HW + s]],
                        axis=1)
                if dj == 0:
                    sh = jnp.where(m_left, sh, zero)
                elif dj == 2:
                    sh = jnp.where(m_right, sh, zero)
                slabs.append(sh)
        patch = jnp.concatenate(slabs, axis=0)
        y = jnp.dot(w_ref[...], patch, preferred_element_type=jnp.float32)
        y2_ref[...] = y.astype(jnp.bfloat16)
        st2_ref[...] = _stats_cols(y)

    return body


def _make_conv3_kernel(count):
    def body(y2_ref, st_ref, g_ref, b_ref, w_ref, y3_ref, st3_ref):
        scale, shift = _finalize(st_ref, g_ref, b_ref, count)
        a = jnp.maximum(y2_ref[...].astype(jnp.float32) * scale + shift, 0.0)
        y = jnp.dot(w_ref[...], a.astype(jnp.bfloat16),
                    preferred_element_type=jnp.float32)
        y3_ref[...] = y.astype(jnp.bfloat16)
        st3_ref[...] = _stats_cols(y)

    return body


def _make_out_kernel(count):
    def body(y3_ref, st_ref, g_ref, b_ref, x_ref, o_ref):
        scale, shift = _finalize(st_ref, g_ref, b_ref, count)
        o_ref[...] = jnp.maximum(
            y3_ref[...].astype(jnp.float32) * scale + shift + x_ref[...], 0.0)

    return body


def kernel(x, w1, w2, w3, g1, b1, g2, b2, g3, b3):
    N, Cin, H, W = x.shape
    C1 = w1.shape[1]
    C2 = w2.shape[2]
    C3 = w3.shape[1]
    HW = H * W
    M = float(N * HW)

    x3 = x.reshape(N, Cin, HW)
    w1t = w1.T.astype(jnp.bfloat16)
    w2t = w2.reshape(9 * C1, C2).T.astype(jnp.bfloat16)
    w3t = w3.T.astype(jnp.bfloat16)
    g1c, b1c = g1.reshape(C1, 1), b1.reshape(C1, 1)
    g2c, b2c = g2.reshape(C2, 1), b2.reshape(C2, 1)
    g3c, b3c = g3.reshape(C3, 1), b3.reshape(C3, 1)

    cp = pltpu.CompilerParams(dimension_semantics=("parallel",),
                              vmem_limit_bytes=_VMEM_LIMIT)

    y1, st1 = pl.pallas_call(
        _conv1_kernel,
        out_shape=(jax.ShapeDtypeStruct((N, C1, HW), jnp.bfloat16),
                   jax.ShapeDtypeStruct((N, C1, 2), jnp.float32)),
        grid=(N,),
        in_specs=[pl.BlockSpec((None, Cin, HW), lambda n: (n, 0, 0)),
                  pl.BlockSpec((C1, Cin), lambda n: (0, 0))],
        out_specs=(pl.BlockSpec((None, C1, HW), lambda n: (n, 0, 0)),
                   pl.BlockSpec((None, C1, 2), lambda n: (n, 0, 0))),
        compiler_params=cp,
        cost_estimate=pl.CostEstimate(
            flops=2 * N * HW * Cin * C1, transcendentals=0,
            bytes_accessed=4 * N * HW * Cin + 2 * N * HW * C1),
    )(x3, w1t)

    y2, st2 = pl.pallas_call(
        _make_conv2_kernel(W, HW, M),
        out_shape=(jax.ShapeDtypeStruct((N, C2, HW), jnp.bfloat16),
                   jax.ShapeDtypeStruct((N, C2, 2), jnp.float32)),
        grid=(N,),
        in_specs=[pl.BlockSpec((None, C1, HW), lambda n: (n, 0, 0)),
                  pl.BlockSpec((N, C1, 2), lambda n: (0, 0, 0)),
                  pl.BlockSpec((C1, 1), lambda n: (0, 0)),
                  pl.BlockSpec((C1, 1), lambda n: (0, 0)),
                  pl.BlockSpec((C2, 9 * C1), lambda n: (0, 0))],
        out_specs=(pl.BlockSpec((None, C2, HW), lambda n: (n, 0, 0)),
                   pl.BlockSpec((None, C2, 2), lambda n: (n, 0, 0))),
        compiler_params=cp,
        cost_estimate=pl.CostEstimate(
            flops=2 * N * HW * 9 * C1 * C2, transcendentals=0,
            bytes_accessed=2 * N * HW * C1 + 2 * N * HW * C2),
    )(y1, st1, g1c, b1c, w2t)

    y3, st3 = pl.pallas_call(
        _make_conv3_kernel(M),
        out_shape=(jax.ShapeDtypeStruct((N, C3, HW), jnp.bfloat16),
                   jax.ShapeDtypeStruct((N, C3, 2), jnp.float32)),
        grid=(N,),
        in_specs=[pl.BlockSpec((None, C2, HW), lambda n: (n, 0, 0)),
                  pl.BlockSpec((N, C2, 2), lambda n: (0, 0, 0)),
                  pl.BlockSpec((C2, 1), lambda n: (0, 0)),
                  pl.BlockSpec((C2, 1), lambda n: (0, 0)),
                  pl.BlockSpec((C3, C2), lambda n: (0, 0))],
        out_specs=(pl.BlockSpec((None, C3, HW), lambda n: (n, 0, 0)),
                   pl.BlockSpec((None, C3, 2), lambda n: (n, 0, 0))),
        compiler_params=cp,
        cost_estimate=pl.CostEstimate(
            flops=2 * N * HW * C2 * C3, transcendentals=0,
            bytes_accessed=2 * N * HW * C2 + 2 * N * HW * C3),
    )(y2, st2, g2c, b2c, w3t)

    out = pl.pallas_call(
        _make_out_kernel(M),
        out_shape=jax.ShapeDtypeStruct((N, C3, HW), jnp.float32),
        grid=(N,),
        in_specs=[pl.BlockSpec((None, C3, HW), lambda n: (n, 0, 0)),
                  pl.BlockSpec((N, C3, 2), lambda n: (0, 0, 0)),
                  pl.BlockSpec((C3, 1), lambda n: (0, 0)),
                  pl.BlockSpec((C3, 1), lambda n: (0, 0)),
                  pl.BlockSpec((None, C3, HW), lambda n: (n, 0, 0))],
        out_specs=pl.BlockSpec((None, C3, HW), lambda n: (n, 0, 0)),
        compiler_params=cp,
        cost_estimate=pl.CostEstimate(
            flops=3 * N * HW * C3, transcendentals=0,
            bytes_accessed=2 * N * HW * C3 + 8 * N * HW * C3),
    )(y3, st3, g3c, b3c, x3)

    return out.reshape(N, C3, H, W)


# speedup vs baseline: 1.9301x; 1.0170x over previous
import jax
import jax.numpy as jnp
from jax import lax
from jax.experimental import pallas as pl
from jax.experimental.pallas import tpu as pltpu

_EPS = 1e-5
_VMEM_LIMIT = 32 * 1024 * 1024

_CONTRACT_00 = (((0,), (0,)), ((), ()))


def _stats_cols(y):
    return jnp.concatenate(
        [jnp.sum(y, axis=1, keepdims=True),
         jnp.sum(y * y, axis=1, keepdims=True)], axis=1)


def _finalize(st_ref, g_ref, b_ref, count):
    tot = jnp.sum(st_ref[...], axis=0)
    mean = tot[:, 0:1] / count
    var = jnp.maximum(tot[:, 1:2] / count - mean * mean, 0.0)
    scale = g_ref[...] * lax.rsqrt(var + _EPS)
    shift = b_ref[...] - mean * scale
    return scale, shift


def _tdot(w, x):
    return lax.dot_general(w, x, _CONTRACT_00,
                           preferred_element_type=jnp.float32)


def _conv1_kernel(x_ref, w_ref, y_ref, st_ref):
    xb = x_ref[...].astype(jnp.bfloat16)
    y = _tdot(w_ref[...].astype(jnp.bfloat16), xb)
    y_ref[...] = y.astype(jnp.bfloat16)
    st_ref[...] = _stats_cols(y)


def _make_conv2_kernel(W, HW, count):
    def body(y1_ref, st_ref, g_ref, b_ref, w_ref, y2_ref, st2_ref):
        C1 = y1_ref.shape[0]
        scale, shift = _finalize(st_ref, g_ref, b_ref, count)
        a = jnp.maximum(y1_ref[...].astype(jnp.float32) * scale + shift, 0.0)
        ab = a.astype(jnp.bfloat16)
        j = lax.broadcasted_iota(jnp.int32, (1, HW), 1) % W
        m_left = j > 0
        m_right = j < W - 1
        zero = jnp.bfloat16(0)
        slabs = []
        for di in range(3):
            for dj in range(3):
                s = (di - 1) * W + (dj - 1)
                if s == 0:
                    sh = ab
                elif s > 0:
                    sh = jnp.concatenate(
                        [ab[:, s:], jnp.zeros((C1, s), ab.dtype)], axis=1)
                else:
                    sh = jnp.concatenate(
                        [jnp.zeros((C1, -s), ab.dtype), ab[:, :HW + s]],
                        axis=1)
                if dj == 0:
                    sh = jnp.where(m_left, sh, zero)
                elif dj == 2:
                    sh = jnp.where(m_right, sh, zero)
                slabs.append(sh)
        patch = jnp.concatenate(slabs, axis=0)
        y = _tdot(w_ref[...].astype(jnp.bfloat16), patch)
        y2_ref[...] = y.astype(jnp.bfloat16)
        st2_ref[...] = _stats_cols(y)

    return body


def _make_conv3_stats_kernel(count):
    def body(y2_ref, st_ref, g_ref, b_ref, w_ref, a2_ref, st3_ref):
        scale, shift = _finalize(st_ref, g_ref, b_ref, count)
        a = jnp.maximum(y2_ref[...].astype(jnp.float32) * scale + shift, 0.0)
        ab = a.astype(jnp.bfloat16)
        a2_ref[...] = ab
        y = _tdot(w_ref[...].astype(jnp.bfloat16), ab)
        st3_ref[...] = _stats_cols(y)

    return body


def _make_out_kernel(count):
    def body(a2_ref, st_ref, g_ref, b_ref, w_ref, x_ref, o_ref):
        scale, shift = _finalize(st_ref, g_ref, b_ref, count)
        y = _tdot(w_ref[...].astype(jnp.bfloat16), a2_ref[...])
        o_ref[...] = jnp.maximum(y * scale + shift + x_ref[...], 0.0)

    return body


def kernel(x, w1, w2, w3, g1, b1, g2, b2, g3, b3):
    N, Cin, H, W = x.shape
    C1 = w1.shape[1]
    C2 = w2.shape[2]
    C3 = w3.shape[1]
    HW = H * W
    M = float(N * HW)

    x3 = x.reshape(N, Cin, HW)
    w2r = w2.reshape(9 * C1, C2)
    g1c, b1c = g1.reshape(C1, 1), b1.reshape(C1, 1)
    g2c, b2c = g2.reshape(C2, 1), b2.reshape(C2, 1)
    g3c, b3c = g3.reshape(C3, 1), b3.reshape(C3, 1)

    cp = pltpu.CompilerParams(dimension_semantics=("parallel",),
                              vmem_limit_bytes=_VMEM_LIMIT)

    y1, st1 = pl.pallas_call(
        _conv1_kernel,
        out_shape=(jax.ShapeDtypeStruct((N, C1, HW), jnp.bfloat16),
                   jax.ShapeDtypeStruct((N, C1, 2), jnp.float32)),
        grid=(N,),
        in_specs=[pl.BlockSpec((None, Cin, HW), lambda n: (n, 0, 0)),
                  pl.BlockSpec((Cin, C1), lambda n: (0, 0))],
        out_specs=(pl.BlockSpec((None, C1, HW), lambda n: (n, 0, 0)),
                   pl.BlockSpec((None, C1, 2), lambda n: (n, 0, 0))),
        compiler_params=cp,
        cost_estimate=pl.CostEstimate(
            flops=2 * N * HW * Cin * C1, transcendentals=0,
            bytes_accessed=4 * N * HW * Cin + 2 * N * HW * C1),
    )(x3, w1)

    y2, st2 = pl.pallas_call(
        _make_conv2_kernel(W, HW, M),
        out_shape=(jax.ShapeDtypeStruct((N, C2, HW), jnp.bfloat16),
                   jax.ShapeDtypeStruct((N, C2, 2), jnp.float32)),
        grid=(N,),
        in_specs=[pl.BlockSpec((None, C1, HW), lambda n: (n, 0, 0)),
                  pl.BlockSpec((N, C1, 2), lambda n: (0, 0, 0)),
                  pl.BlockSpec((C1, 1), lambda n: (0, 0)),
                  pl.BlockSpec((C1, 1), lambda n: (0, 0)),
                  pl.BlockSpec((9 * C1, C2), lambda n: (0, 0))],
        out_specs=(pl.BlockSpec((None, C2, HW), lambda n: (n, 0, 0)),
                   pl.BlockSpec((None, C2, 2), lambda n: (n, 0, 0))),
        compiler_params=cp,
        cost_estimate=pl.CostEstimate(
            flops=2 * N * HW * 9 * C1 * C2, transcendentals=0,
            bytes_accessed=2 * N * HW * C1 + 2 * N * HW * C2),
    )(y1, st1, g1c, b1c, w2r)

    a2, st3 = pl.pallas_call(
        _make_conv3_stats_kernel(M),
        out_shape=(jax.ShapeDtypeStruct((N, C2, HW), jnp.bfloat16),
                   jax.ShapeDtypeStruct((N, C3, 2), jnp.float32)),
        grid=(N,),
        in_specs=[pl.BlockSpec((None, C2, HW), lambda n: (n, 0, 0)),
                  pl.BlockSpec((N, C2, 2), lambda n: (0, 0, 0)),
                  pl.BlockSpec((C2, 1), lambda n: (0, 0)),
                  pl.BlockSpec((C2, 1), lambda n: (0, 0)),
                  pl.BlockSpec((C2, C3), lambda n: (0, 0))],
        out_specs=(pl.BlockSpec((None, C2, HW), lambda n: (n, 0, 0)),
                   pl.BlockSpec((None, C3, 2), lambda n: (n, 0, 0))),
        compiler_params=cp,
        cost_estimate=pl.CostEstimate(
            flops=2 * N * HW * C2 * C3, transcendentals=0,
            bytes_accessed=2 * N * HW * C2 + 2 * N * HW * C2),
    )(y2, st2, g2c, b2c, w3)

    out = pl.pallas_call(
        _make_out_kernel(M),
        out_shape=jax.ShapeDtypeStruct((N, C3, HW), jnp.float32),
        grid=(N,),
        in_specs=[pl.BlockSpec((None, C2, HW), lambda n: (n, 0, 0)),
                  pl.BlockSpec((N, C3, 2), lambda n: (0, 0, 0)),
                  pl.BlockSpec((C3, 1), lambda n: (0, 0)),
                  pl.BlockSpec((C3, 1), lambda n: (0, 0)),
                  pl.BlockSpec((C2, C3), lambda n: (0, 0)),
                  pl.BlockSpec((None, C3, HW), lambda n: (n, 0, 0))],
        out_specs=pl.BlockSpec((None, C3, HW), lambda n: (n, 0, 0)),
        compiler_params=cp,
        cost_estimate=pl.CostEstimate(
            flops=2 * N * HW * C2 * C3 + 3 * N * HW * C3, transcendentals=0,
            bytes_accessed=2 * N * HW * C2 + 8 * N * HW * C3),
    )(a2, st3, g3c, b3c, w3, x3)

    return out.reshape(N, C3, H, W)


# speedup vs baseline: 3.2980x; 1.7087x over previous
import jax
import jax.numpy as jnp
from jax import lax
from jax.experimental import pallas as pl
from jax.experimental.pallas import tpu as pltpu

_EPS = 1e-5
_VMEM_LIMIT = 32 * 1024 * 1024


def _stats_rows(y):
    return jnp.concatenate(
        [jnp.sum(y, axis=0, keepdims=True),
         jnp.sum(y * y, axis=0, keepdims=True)], axis=0)


def _finalize(st_ref, g_ref, b_ref, count):
    tot = jnp.sum(st_ref[...], axis=0)
    mean = tot[0:1] / count
    var = jnp.maximum(tot[1:2] / count - mean * mean, 0.0)
    scale = g_ref[...] * lax.rsqrt(var + _EPS)
    shift = b_ref[...] - mean * scale
    return scale, shift


def _conv1_kernel(x_ref, w_ref, y_ref, st_ref):
    xb = x_ref[...].astype(jnp.bfloat16)
    y = jnp.dot(xb, w_ref[...].astype(jnp.bfloat16),
                preferred_element_type=jnp.float32)
    y_ref[...] = y.astype(jnp.bfloat16)
    st_ref[...] = _stats_rows(y)


def _make_conv2_kernel(W, HW, count):
    def body(y1_ref, st_ref, g_ref, b_ref, w_ref, y2_ref, st2_ref):
        C1 = y1_ref.shape[-1]
        scale, shift = _finalize(st_ref, g_ref, b_ref, count)
        a = jnp.maximum(y1_ref[...].astype(jnp.float32) * scale + shift, 0.0)
        ab = a.astype(jnp.bfloat16)
        i = lax.broadcasted_iota(jnp.int32, (HW, 1), 0) % W
        m_left = i > 0
        m_right = i < W - 1
        zero = jnp.bfloat16(0)
        slabs = []
        for di in range(3):
            for dj in range(3):
                s = (di - 1) * W + (dj - 1)
                if s == 0:
                    sh = ab
                elif s > 0:
                    sh = jnp.concatenate(
                        [ab[s:], jnp.zeros((s, C1), ab.dtype)], axis=0)
                else:
                    sh = jnp.concatenate(
                        [jnp.zeros((-s, C1), ab.dtype), ab[:HW + s]], axis=0)
                if dj == 0:
                    sh = jnp.where(m_left, sh, zero)
                elif dj == 2:
                    sh = jnp.where(m_right, sh, zero)
                slabs.append(sh)
        patch = jnp.concatenate(slabs, axis=1)
        y = jnp.dot(patch, w_ref[...].astype(jnp.bfloat16),
                    preferred_element_type=jnp.float32)
        y2_ref[...] = y.astype(jnp.bfloat16)
        st2_ref[...] = _stats_rows(y)

    return body


def _make_conv3_stats_kernel(count):
    def body(y2_ref, st_ref, g_ref, b_ref, w_ref, a2_ref, st3_ref):
        scale, shift = _finalize(st_ref, g_ref, b_ref, count)
        a = jnp.maximum(y2_ref[...].astype(jnp.float32) * scale + shift, 0.0)
        ab = a.astype(jnp.bfloat16)
        a2_ref[...] = ab
        y = jnp.dot(ab, w_ref[...].astype(jnp.bfloat16),
                    preferred_element_type=jnp.float32)
        st3_ref[...] = _stats_rows(y)

    return body


def _make_out_kernel(count):
    def body(a2_ref, st_ref, g_ref, b_ref, w_ref, x_ref, o_ref):
        scale, shift = _finalize(st_ref, g_ref, b_ref, count)
        y = jnp.dot(a2_ref[...], w_ref[...].astype(jnp.bfloat16),
                    preferred_element_type=jnp.float32)
        o_ref[...] = jnp.maximum(y * scale + shift + x_ref[...], 0.0)

    return body


def kernel(x, w1, w2, w3, g1, b1, g2, b2, g3, b3):
    N, Cin, H, W = x.shape
    C1 = w1.shape[1]
    C2 = w2.shape[2]
    C3 = w3.shape[1]
    HW = H * W
    M = N * HW
    Mf = float(M)

    x2d = jnp.transpose(x, (0, 2, 3, 1)).reshape(M, Cin)
    w2r = w2.reshape(9 * C1, C2)

    cp = pltpu.CompilerParams(dimension_semantics=("parallel",),
                              vmem_limit_bytes=_VMEM_LIMIT)

    y1, st1 = pl.pallas_call(
        _conv1_kernel,
        out_shape=(jax.ShapeDtypeStruct((N, HW, C1), jnp.bfloat16),
                   jax.ShapeDtypeStruct((N, 2, C1), jnp.float32)),
        grid=(N,),
        in_specs=[pl.BlockSpec((HW, Cin), lambda n: (n, 0)),
                  pl.BlockSpec((Cin, C1), lambda n: (0, 0))],
        out_specs=(pl.BlockSpec((None, HW, C1), lambda n: (n, 0, 0)),
                   pl.BlockSpec((None, 2, C1), lambda n: (n, 0, 0))),
        compiler_params=cp,
        cost_estimate=pl.CostEstimate(
            flops=2 * M * Cin * C1, transcendentals=0,
            bytes_accessed=4 * M * Cin + 2 * M * C1),
    )(x2d, w1)

    y2, st2 = pl.pallas_call(
        _make_conv2_kernel(W, HW, Mf),
        out_shape=(jax.ShapeDtypeStruct((N, HW, C2), jnp.bfloat16),
                   jax.ShapeDtypeStruct((N, 2, C2), jnp.float32)),
        grid=(N,),
        in_specs=[pl.BlockSpec((None, HW, C1), lambda n: (n, 0, 0)),
                  pl.BlockSpec((N, 2, C1), lambda n: (0, 0, 0)),
                  pl.BlockSpec((1, C1), lambda n: (0, 0)),
                  pl.BlockSpec((1, C1), lambda n: (0, 0)),
                  pl.BlockSpec((9 * C1, C2), lambda n: (0, 0))],
        out_specs=(pl.BlockSpec((None, HW, C2), lambda n: (n, 0, 0)),
                   pl.BlockSpec((None, 2, C2), lambda n: (n, 0, 0))),
        compiler_params=cp,
        cost_estimate=pl.CostEstimate(
            flops=2 * M * 9 * C1 * C2, transcendentals=0,
            bytes_accessed=2 * M * C1 + 2 * M * C2),
    )(y1, st1, g1, b1, w2r)

    a2, st3 = pl.pallas_call(
        _make_conv3_stats_kernel(Mf),
        out_shape=(jax.ShapeDtypeStruct((N, HW, C2), jnp.bfloat16),
                   jax.ShapeDtypeStruct((N, 2, C3), jnp.float32)),
        grid=(N,),
        in_specs=[pl.BlockSpec((None, HW, C2), lambda n: (n, 0, 0)),
                  pl.BlockSpec((N, 2, C2), lambda n: (0, 0, 0)),
                  pl.BlockSpec((1, C2), lambda n: (0, 0)),
                  pl.BlockSpec((1, C2), lambda n: (0, 0)),
                  pl.BlockSpec((C2, C3), lambda n: (0, 0))],
        out_specs=(pl.BlockSpec((None, HW, C2), lambda n: (n, 0, 0)),
                   pl.BlockSpec((None, 2, C3), lambda n: (n, 0, 0))),
        compiler_params=cp,
        cost_estimate=pl.CostEstimate(
            flops=2 * M * C2 * C3, transcendentals=0,
            bytes_accessed=2 * M * C2 + 2 * M * C2),
    )(y2, st2, g2, b2, w3)

    out2d = pl.pallas_call(
        _make_out_kernel(Mf),
        out_shape=jax.ShapeDtypeStruct((M, C3), jnp.float32),
        grid=(N,),
        in_specs=[pl.BlockSpec((None, HW, C2), lambda n: (n, 0, 0)),
                  pl.BlockSpec((N, 2, C3), lambda n: (0, 0, 0)),
                  pl.BlockSpec((1, C3), lambda n: (0, 0)),
                  pl.BlockSpec((1, C3), lambda n: (0, 0)),
                  pl.BlockSpec((C2, C3), lambda n: (0, 0)),
                  pl.BlockSpec((HW, C3), lambda n: (n, 0))],
        out_specs=pl.BlockSpec((HW, C3), lambda n: (n, 0)),
        compiler_params=cp,
        cost_estimate=pl.CostEstimate(
            flops=2 * M * C2 * C3 + 3 * M * C3, transcendentals=0,
            bytes_accessed=2 * M * C2 + 8 * M * C3),
    )(a2, st3, g3, b3, w3, x2d)

    return jnp.transpose(out2d.reshape(N, H, W, C3), (0, 3, 1, 2))


# speedup vs baseline: 3.6578x; 1.1091x over previous
import jax
import jax.numpy as jnp
from jax import lax
from jax.experimental import pallas as pl
from jax.experimental.pallas import tpu as pltpu

_EPS = 1e-5
_VMEM_LIMIT = 32 * 1024 * 1024


def _stats_rows(y):
    return jnp.concatenate(
        [jnp.sum(y, axis=0, keepdims=True),
         jnp.sum(y * y, axis=0, keepdims=True)], axis=0)


def _finalize(st_ref, g_ref, b_ref, count):
    tot = jnp.sum(st_ref[...], axis=0)
    mean = tot[0:1] / count
    var = jnp.maximum(tot[1:2] / count - mean * mean, 0.0)
    scale = g_ref[...] * lax.rsqrt(var + _EPS)
    shift = b_ref[...] - mean * scale
    return scale, shift


def _conv1_kernel(x_ref, w_ref, y_ref, st_ref):
    xb = x_ref[...].astype(jnp.bfloat16)
    y = jnp.dot(xb, w_ref[...].astype(jnp.bfloat16),
                preferred_element_type=jnp.float32)
    y_ref[...] = y.astype(jnp.bfloat16)
    st_ref[...] = _stats_rows(y)


def _make_conv2_kernel(W, HW, count):
    def body(y1_ref, st_ref, g_ref, b_ref, cm_ref, w_ref, y2_ref, st2_ref):
        C1 = y1_ref.shape[-1]
        scale, shift = _finalize(st_ref, g_ref, b_ref, count)
        a = jnp.maximum(y1_ref[...].astype(jnp.float32) * scale + shift, 0.0)
        ab = a.astype(jnp.bfloat16)
        cm = cm_ref[...]
        src_by_dj = (ab * cm[:, 1:2], ab, ab * cm[:, 0:1])
        slabs = []
        for di in range(3):
            for dj in range(3):
                s = (di - 1) * W + (dj - 1)
                src = src_by_dj[dj]
                if s == 0:
                    sh = src
                elif s > 0:
                    sh = jnp.concatenate(
                        [src[s:], jnp.zeros((s, C1), src.dtype)], axis=0)
                else:
                    sh = jnp.concatenate(
                        [jnp.zeros((-s, C1), src.dtype), src[:HW + s]],
                        axis=0)
                slabs.append(sh)
        patch = jnp.concatenate(slabs, axis=1)
        y = jnp.dot(patch, w_ref[...].astype(jnp.bfloat16),
                    preferred_element_type=jnp.float32)
        y2_ref[...] = y.astype(jnp.bfloat16)
        st2_ref[...] = _stats_rows(y)

    return body


def _make_conv3_stats_kernel(count):
    def body(y2_ref, st_ref, g_ref, b_ref, w_ref, a2_ref, st3_ref):
        scale, shift = _finalize(st_ref, g_ref, b_ref, count)
        a = jnp.maximum(y2_ref[...].astype(jnp.float32) * scale + shift, 0.0)
        ab = a.astype(jnp.bfloat16)
        a2_ref[...] = ab
        y = jnp.dot(ab, w_ref[...].astype(jnp.bfloat16),
                    preferred_element_type=jnp.float32)
        st3_ref[...] = _stats_rows(y)

    return body


def _make_out_kernel(count):
    def body(a2_ref, st_ref, g_ref, b_ref, w_ref, x_ref, o_ref):
        scale, shift = _finalize(st_ref, g_ref, b_ref, count)
        y = jnp.dot(a2_ref[...], w_ref[...].astype(jnp.bfloat16),
                    preferred_element_type=jnp.float32)
        o_ref[...] = jnp.maximum(y * scale + shift + x_ref[...], 0.0)

    return body


def kernel(x, w1, w2, w3, g1, b1, g2, b2, g3, b3):
    N, Cin, H, W = x.shape
    C1 = w1.shape[1]
    C2 = w2.shape[2]
    C3 = w3.shape[1]
    HW = H * W
    M = N * HW
    Mf = float(M)

    x2d = jnp.transpose(x, (0, 2, 3, 1)).reshape(M, Cin)
    w2r = w2.reshape(9 * C1, C2)
    j_idx = jnp.arange(HW, dtype=jnp.int32) % W
    col_mask = jnp.stack([j_idx > 0, j_idx < W - 1],
                         axis=1).astype(jnp.bfloat16)

    cp = pltpu.CompilerParams(dimension_semantics=("parallel",),
                              vmem_limit_bytes=_VMEM_LIMIT)

    y1, st1 = pl.pallas_call(
        _conv1_kernel,
        out_shape=(jax.ShapeDtypeStruct((N, HW, C1), jnp.bfloat16),
                   jax.ShapeDtypeStruct((N, 2, C1), jnp.float32)),
        grid=(N,),
        in_specs=[pl.BlockSpec((HW, Cin), lambda n: (n, 0)),
                  pl.BlockSpec((Cin, C1), lambda n: (0, 0))],
        out_specs=(pl.BlockSpec((None, HW, C1), lambda n: (n, 0, 0)),
                   pl.BlockSpec((None, 2, C1), lambda n: (n, 0, 0))),
        compiler_params=cp,
        cost_estimate=pl.CostEstimate(
            flops=2 * M * Cin * C1, transcendentals=0,
            bytes_accessed=4 * M * Cin + 2 * M * C1),
    )(x2d, w1)

    y2, st2 = pl.pallas_call(
        _make_conv2_kernel(W, HW, Mf),
        out_shape=(jax.ShapeDtypeStruct((N, HW, C2), jnp.bfloat16),
                   jax.ShapeDtypeStruct((N, 2, C2), jnp.float32)),
        grid=(N,),
        in_specs=[pl.BlockSpec((None, HW, C1), lambda n: (n, 0, 0)),
                  pl.BlockSpec((N, 2, C1), lambda n: (0, 0, 0)),
                  pl.BlockSpec((1, C1), lambda n: (0, 0)),
                  pl.BlockSpec((1, C1), lambda n: (0, 0)),
                  pl.BlockSpec((HW, 2), lambda n: (0, 0)),
                  pl.BlockSpec((9 * C1, C2), lambda n: (0, 0))],
        out_specs=(pl.BlockSpec((None, HW, C2), lambda n: (n, 0, 0)),
                   pl.BlockSpec((None, 2, C2), lambda n: (n, 0, 0))),
        compiler_params=cp,
        cost_estimate=pl.CostEstimate(
            flops=2 * M * 9 * C1 * C2, transcendentals=0,
            bytes_accessed=2 * M * C1 + 2 * M * C2),
    )(y1, st1, g1, b1, col_mask, w2r)

    a2, st3 = pl.pallas_call(
        _make_conv3_stats_kernel(Mf),
        out_shape=(jax.ShapeDtypeStruct((N, HW, C2), jnp.bfloat16),
                   jax.ShapeDtypeStruct((N, 2, C3), jnp.float32)),
        grid=(N,),
        in_specs=[pl.BlockSpec((None, HW, C2), lambda n: (n, 0, 0)),
                  pl.BlockSpec((N, 2, C2), lambda n: (0, 0, 0)),
                  pl.BlockSpec((1, C2), lambda n: (0, 0)),
                  pl.BlockSpec((1, C2), lambda n: (0, 0)),
                  pl.BlockSpec((C2, C3), lambda n: (0, 0))],
        out_specs=(pl.BlockSpec((None, HW, C2), lambda n: (n, 0, 0)),
                   pl.BlockSpec((None, 2, C3), lambda n: (n, 0, 0))),
        compiler_params=cp,
        cost_estimate=pl.CostEstimate(
            flops=2 * M * C2 * C3, transcendentals=0,
            bytes_accessed=2 * M * C2 + 2 * M * C2),
    )(y2, st2, g2, b2, w3)

    out2d = pl.pallas_call(
        _make_out_kernel(Mf),
        out_shape=jax.ShapeDtypeStruct((M, C3), jnp.float32),
        grid=(N,),
        in_specs=[pl.BlockSpec((None, HW, C2), lambda n: (n, 0, 0)),
                  pl.BlockSpec((N, 2, C3), lambda n: (0, 0, 0)),
                  pl.BlockSpec((1, C3), lambda n: (0, 0)),
                  pl.BlockSpec((1, C3), lambda n: (0, 0)),
                  pl.BlockSpec((C2, C3), lambda n: (0, 0)),
                  pl.BlockSpec((HW, C3), lambda n: (n, 0))],
        out_specs=pl.BlockSpec((HW, C3), lambda n: (n, 0)),
        compiler_params=cp,
        cost_estimate=pl.CostEstimate(
            flops=2 * M * C2 * C3 + 3 * M * C3, transcendentals=0,
            bytes_accessed=2 * M * C2 + 8 * M * C3),
    )(a2, st3, g3, b3, w3, x2d)

    return jnp.transpose(out2d.reshape(N, H, W, C3), (0, 3, 1, 2))
